# baseline (device time: 33546 ns/iter reference)
import jax
import jax.numpy as jnp
from jax import lax
from jax.experimental import pallas as pl
from jax.experimental.pallas import tpu as pltpu

N_DEV = 4
SQ = 256
SKV = 4096
HQ = 32
DH = 128
H_LOC = HQ // N_DEV
QB = 4
BLK = 64
T = SKV // BLK // QB
NU = 2 * QB
UB = BLK // 2
D_MODEL = 1024
SCALE = 0.08838834764831843
F32 = jnp.float32


def kernel(x, Wq, K_ext, V_ext, Wo):
    K5 = K_ext.reshape(T, QB, BLK, HQ, DH)
    V5 = V_ext.reshape(T, QB, BLK, HQ, DH)

    def body(x_ref, wq_hbm, k_hbm, v_hbm, wo_hbm, out_ref,
             k_buf, v_buf, wq_ref, wo_ref, recv1, recv2,
             k_sems, v_sems, w_sems, s1_sems, r1_sems, s2_sems, r2_sems):
        my_i = lax.axis_index("i")
        h0 = my_i * H_LOC
        p1 = my_i + 1 - 2 * (my_i % 2)
        p2 = (N_DEV - 1) - my_i

        cwq = pltpu.make_async_copy(wq_hbm, wq_ref, w_sems.at[0])
        cwq.start()
        copies = []
        for qb in range(QB):
            for h in range(H_LOC):
                ck = pltpu.make_async_copy(
                    k_hbm.at[:, qb, :, h0 + h, :],
                    k_buf.at[qb, h],
                    k_sems.at[qb, h],
                )
                cv = pltpu.make_async_copy(
                    v_hbm.at[:, qb, :, h0 + h, :],
                    v_buf.at[qb, h],
                    v_sems.at[qb, h],
                )
                ck.start()
                cv.start()
                copies.append((ck, cv))
            if qb == 0:
                cwo = pltpu.make_async_copy(wo_hbm, wo_ref, w_sems.at[1])
                cwo.start()

        barrier_sem = pltpu.get_barrier_semaphore()
        for nbr in (p1, p2):
            pl.semaphore_signal(
                barrier_sem, inc=1,
                device_id=(nbr,), device_id_type=pl.DeviceIdType.MESH,
            )
        pl.semaphore_wait(barrier_sem, 2)

        cwq.wait()
        q_all = jnp.dot(x_ref[0], wq_ref[:, :], preferred_element_type=F32)
        wo_waited = []

        def compute_chunk(qb):
            ctxs = []
            for h in range(H_LOC):
                ck, cv = copies[qb * H_LOC + h]
                ck.wait()
                cv.wait()
                q = q_all[qb * BLK:(qb + 1) * BLK, h * DH:(h + 1) * DH]
                kmat = k_buf[qb, h].reshape(T * BLK, DH)
                vmat = v_buf[qb, h].reshape(T * BLK, DH)
                s = lax.dot_general(
                    q, kmat, (((1,), (1,)), ((), ())),
                    preferred_element_type=F32,
                ) * SCALE
                e = jnp.exp(s)
                ctx_u = lax.dot_general(
                    e, vmat, (((1,), (0,)), ((), ())),
                    preferred_element_type=F32,
                )
                denom = jnp.sum(e, axis=1, keepdims=True)
                ctxs.append(ctx_u * (1.0 / denom))
            ctx_c = jnp.concatenate(ctxs, axis=1)
            if not wo_waited:
                cwo.wait()
                wo_waited.append(True)
            out_ref[0, qb * BLK:(qb + 1) * BLK, :] = jnp.dot(
                ctx_c, wo_ref[:, :], preferred_element_type=F32
            )

        def exch_start(u, partner, dst, ssem, rsem):
            r = pltpu.make_async_remote_copy(
                src_ref=out_ref.at[0, pl.ds(u * UB, UB)],
                dst_ref=dst.at[u],
                send_sem=ssem.at[u],
                recv_sem=rsem.at[u],
                device_id=(partner,),
                device_id_type=pl.DeviceIdType.MESH,
            )
            r.start()
            return r

        def exch_finish(u, r, src):
            r.wait()
            out_ref[0, u * UB:(u + 1) * UB, :] = (
                out_ref[0, u * UB:(u + 1) * UB, :] + src[u]
            )

        s1 = {}
        s2 = {}
        for c in range(QB):
            compute_chunk(c)
            for u in (2 * c, 2 * c + 1):
                s1[u] = exch_start(u, p1, recv1, s1_sems, r1_sems)
            if c >= 1:
                for u in (2 * c - 2, 2 * c - 1):
                    exch_finish(u, s1[u], recv1)
                    s2[u] = exch_start(u, p2, recv2, s2_sems, r2_sems)
        for u in (NU - 2, NU - 1):
            exch_finish(u, s1[u], recv1)
            s2[u] = exch_start(u, p2, recv2, s2_sems, r2_sems)
        for u in range(NU):
            exch_finish(u, s2[u], recv2)

    return pl.pallas_call(
        body,
        out_shape=jax.ShapeDtypeStruct((1, SQ, D_MODEL), F32),
        in_specs=[
            pl.BlockSpec(memory_space=pltpu.MemorySpace.VMEM),
            pl.BlockSpec(memory_space=pltpu.MemorySpace.HBM),
            pl.BlockSpec(memory_space=pltpu.MemorySpace.HBM),
            pl.BlockSpec(memory_space=pltpu.MemorySpace.HBM),
            pl.BlockSpec(memory_space=pltpu.MemorySpace.HBM),
        ],
        out_specs=pl.BlockSpec(memory_space=pltpu.MemorySpace.VMEM),
        scratch_shapes=[
            pltpu.VMEM((QB, H_LOC, T, BLK, DH), F32),
            pltpu.VMEM((QB, H_LOC, T, BLK, DH), F32),
            pltpu.VMEM((D_MODEL, D_MODEL), F32),
            pltpu.VMEM((D_MODEL, D_MODEL), F32),
            pltpu.VMEM((NU, UB, D_MODEL), F32),
            pltpu.VMEM((NU, UB, D_MODEL), F32),
            pltpu.SemaphoreType.DMA((QB, H_LOC)),
            pltpu.SemaphoreType.DMA((QB, H_LOC)),
            pltpu.SemaphoreType.DMA((2,)),
            pltpu.SemaphoreType.DMA((NU,)),
            pltpu.SemaphoreType.DMA((NU,)),
            pltpu.SemaphoreType.DMA((NU,)),
            pltpu.SemaphoreType.DMA((NU,)),
        ],
        compiler_params=pltpu.CompilerParams(
            collective_id=0,
            vmem_limit_bytes=60 * 1024 * 1024,
        ),
    )(x, Wq, K5, V5, Wo)
